# Initial kernel scaffold; baseline (speedup 1.0000x reference)
import jax
import jax.numpy as jnp
from jax.experimental import pallas as pl


def kernel(x, buffer, position, length):
    raise NotImplementedError("write your pallas kernel here")


# speedup vs baseline: 1.1266x; 1.1266x over previous
import functools

import jax
import jax.numpy as jnp
from jax.experimental import pallas as pl
from jax.experimental.pallas import tpu as pltpu

MAXLEN = 4096
DIM = 2048
NROWS = 8192

RBLK = 512
CBLK = 512


def _reduce_body(x_ref, acc_ref):
    i = pl.program_id(0)

    @pl.when(i == 0)
    def _():
        acc_ref[...] = jnp.zeros_like(acc_ref)

    acc_ref[...] += jnp.sum(x_ref[...], axis=0, keepdims=True)

    @pl.when(i == pl.num_programs(0) - 1)
    def _():
        acc_ref[...] *= (1.0 / NROWS)


def _copy_scatter_body(pos_ref, buf_ref, mean_ref, out_ref):
    i = pl.program_id(0)
    out_ref[...] = buf_ref[...]
    local = pos_ref[0] - i * CBLK

    @pl.when((local >= 0) & (local < CBLK))
    def _():
        out_ref[pl.ds(local, 1), :] = mean_ref[...]


def kernel(x, buffer, position, length):
    del length
    pos = jnp.asarray(position, jnp.int32).reshape(1)

    mean = pl.pallas_call(
        _reduce_body,
        grid=(NROWS // RBLK,),
        in_specs=[pl.BlockSpec((RBLK, DIM), lambda i: (i, 0))],
        out_specs=pl.BlockSpec((1, DIM), lambda i: (0, 0)),
        out_shape=jax.ShapeDtypeStruct((1, DIM), jnp.float32),
    )(x)

    new_buffer = pl.pallas_call(
        _copy_scatter_body,
        grid_spec=pltpu.PrefetchScalarGridSpec(
            num_scalar_prefetch=1,
            grid=(MAXLEN // CBLK,),
            in_specs=[
                pl.BlockSpec((CBLK, DIM), lambda i, p: (i, 0)),
                pl.BlockSpec((1, DIM), lambda i, p: (0, 0)),
            ],
            out_specs=pl.BlockSpec((CBLK, DIM), lambda i, p: (i, 0)),
        ),
        out_shape=jax.ShapeDtypeStruct((MAXLEN, DIM), jnp.float32),
    )(pos, buffer, mean)

    return new_buffer


# speedup vs baseline: 1.1377x; 1.0098x over previous
import jax
import jax.numpy as jnp
from jax.experimental import pallas as pl
from jax.experimental.pallas import tpu as pltpu

MAXLEN = 4096
DIM = 2048
NROWS = 8192

GRID = 16
RBLK = NROWS // GRID
CBLK = MAXLEN // GRID


def _perm(i, pos_ref):
    b_pos = pos_ref[0] // CBLK
    return jnp.where(i == GRID - 1, b_pos, i + (i >= b_pos).astype(i.dtype))


def _body(pos_ref, x_ref, buf_ref, out_ref, acc_ref):
    i = pl.program_id(0)

    @pl.when(i == 0)
    def _():
        acc_ref[...] = jnp.zeros_like(acc_ref)

    acc_ref[...] += jnp.sum(x_ref[...], axis=0, keepdims=True)
    out_ref[...] = buf_ref[...]

    @pl.when(i == GRID - 1)
    def _():
        local = pos_ref[0] % CBLK
        out_ref[pl.ds(local, 1), :] = acc_ref[...] * (1.0 / NROWS)


def kernel(x, buffer, position, length):
    del length
    pos = jnp.asarray(position, jnp.int32).reshape(1)

    new_buffer = pl.pallas_call(
        _body,
        grid_spec=pltpu.PrefetchScalarGridSpec(
            num_scalar_prefetch=1,
            grid=(GRID,),
            in_specs=[
                pl.BlockSpec((RBLK, DIM), lambda i, p: (i, 0)),
                pl.BlockSpec((CBLK, DIM), lambda i, p: (_perm(i, p), 0)),
            ],
            out_specs=pl.BlockSpec((CBLK, DIM), lambda i, p: (_perm(i, p), 0)),
            scratch_shapes=[pltpu.VMEM((1, DIM), jnp.float32)],
        ),
        out_shape=jax.ShapeDtypeStruct((MAXLEN, DIM), jnp.float32),
    )(pos, x, buffer)

    return new_buffer
